# Initial kernel scaffold; baseline (speedup 1.0000x reference)
import jax
import jax.numpy as jnp
from jax.experimental import pallas as pl


def kernel(step, z_loga, eps):
    raise NotImplementedError("write your pallas kernel here")


# speedup vs baseline: 80.9699x; 80.9699x over previous
import functools

import jax
import jax.numpy as jnp
from jax import lax
from jax.experimental import pallas as pl
from jax.experimental.pallas import tpu as pltpu

_ROWS = 128
_COLS = 32768
_K = 16384
_TEMP = 2.0 / 3.0
_ROW_BLOCK = 16


def _mask_kernel(z_ref, eps_ref, out_ref, keys_ref):
    eps = jnp.clip(eps_ref[...], 1e-6, 1.0 - 1e-6)
    gumbel = -jnp.log(-jnp.log(eps))
    s = z_ref[...] + gumbel
    b = lax.bitcast_convert_type(s, jnp.uint32)
    sign = (b >> 31).astype(jnp.bool_)
    keys = jnp.where(sign, ~b, b | jnp.uint32(0x80000000))
    keys_ref[...] = keys

    def step(i, t):
        bit = jnp.uint32(0x80000000) >> i
        cand = t | bit
        cnt = jnp.sum((keys_ref[...] >= cand).astype(jnp.int32), axis=1,
                      keepdims=True)
        return jnp.where(cnt >= _K, cand, t)

    t0 = jnp.zeros((_ROW_BLOCK, 1), dtype=jnp.uint32)
    t = lax.fori_loop(0, 32, step, t0)
    out_ref[...] = (keys_ref[...] >= t).astype(jnp.float32)


@jax.jit
def kernel(step, z_loga, eps):
    del step
    grid = (_ROWS // _ROW_BLOCK,)
    spec = pl.BlockSpec((_ROW_BLOCK, _COLS), lambda i: (i, 0))
    out = pl.pallas_call(
        _mask_kernel,
        grid=grid,
        in_specs=[spec, spec],
        out_specs=spec,
        out_shape=jax.ShapeDtypeStruct((_ROWS, _COLS), jnp.float32),
        scratch_shapes=[pltpu.VMEM((_ROW_BLOCK, _COLS), jnp.uint32)],
    )(z_loga, eps)
    return out
